# baseline (speedup 1.0000x reference)
import jax
import jax.numpy as jnp
from jax import lax
from jax.experimental import pallas as pl
from jax.experimental.pallas import tpu as pltpu
from jax.experimental.pallas import tpu_sc as plsc

_N = 10000
_E = 320000
_D = 128
_T = 4
_STEPS = 8

_NC = 2
_NS = 16
_NW = _NC * _NS
_LANES = 128
_CHUNKS = 80
_EPAD = _NW * _CHUNKS * _LANES
_NPAD = 11136
_RPT = _NPAD // _NS
_GRP = 16
_NGRP = _CHUNKS // _GRP
_FULL_TILES = _N // _RPT
_TAIL = _N - _FULL_TILES * _RPT

_B = 1000
_NB = _N // _B


_sc_mesh = plsc.VectorSubcoreMesh(core_axis_name="c", subcore_axis_name="s")


def _agg_group(table, accum, gv, lv, rows0, rows1, semg0, semg1, sems0, sems1,
               primed):
    if not primed:
        pltpu.async_copy(table.at[gv.at[0]], rows0, semg0)
        pltpu.async_copy(table.at[gv.at[1]], rows1, semg1)

    def body(k, carry):
        j0 = 2 * k
        j1 = j0 + 1
        jn0 = jnp.minimum(j0 + 2, _GRP - 2)
        jn1 = jnp.minimum(j1 + 2, _GRP - 1)
        pltpu.make_async_copy(table.at[gv.at[j0]], rows0, semg0).wait()
        pltpu.async_copy(rows0, accum.at[lv.at[j0]], sems0, add=True)
        pltpu.make_async_copy(table.at[gv.at[j1]], rows1, semg1).wait()
        pltpu.async_copy(rows1, accum.at[lv.at[j1]], sems1, add=True)
        pltpu.make_async_copy(rows0, accum.at[lv.at[j0]], sems0).wait()
        pltpu.async_copy(table.at[gv.at[jn0]], rows0, semg0)
        pltpu.make_async_copy(rows1, accum.at[lv.at[j1]], sems1).wait()
        pltpu.async_copy(table.at[gv.at[jn1]], rows1, semg1)
        return carry

    lax.fori_loop(0, _GRP // 2, body, 0)
    pltpu.make_async_copy(table.at[gv.at[_GRP - 2]], rows0, semg0).wait()
    pltpu.make_async_copy(table.at[gv.at[_GRP - 1]], rows1, semg1).wait()


def _agg_body(table, gidx, ldst, zeros, out, accum, gvA, lvA, gvB, lvB,
              rows0, rows1, semg0, semg1, sems0, sems1, semA, semB, semz):
    c = lax.axis_index("c")
    s = lax.axis_index("s")
    w = c * _NS + s

    pltpu.async_copy(zeros.at[pl.ds(s * _RPT, _RPT)],
                     accum.at[pl.ds(s * _RPT, _RPT)], semz)

    idx_bufs = [(gvA, lvA, semA), (gvB, lvB, semB)]

    def fetch(buf, g):
        gv, lv, sem = buf
        pltpu.async_copy(gidx.at[w, pl.ds(g * _GRP, _GRP)], gv, sem)
        pltpu.async_copy(ldst.at[w, pl.ds(g * _GRP, _GRP)], lv, sem)

    def wait_fetch(buf, g):
        gv, lv, sem = buf
        pltpu.make_async_copy(gidx.at[w, pl.ds(g * _GRP, _GRP)], gv, sem).wait()
        pltpu.make_async_copy(ldst.at[w, pl.ds(g * _GRP, _GRP)], lv, sem).wait()

    fetch(idx_bufs[0], 0)
    wait_fetch(idx_bufs[0], 0)
    pltpu.async_copy(table.at[gvA.at[0]], rows0, semg0)
    pltpu.async_copy(table.at[gvA.at[1]], rows1, semg1)
    pltpu.make_async_copy(zeros.at[pl.ds(s * _RPT, _RPT)],
                          accum.at[pl.ds(s * _RPT, _RPT)], semz).wait()
    plsc.subcore_barrier()

    for g in range(_NGRP):
        buf = idx_bufs[g % 2]
        if g + 1 < _NGRP:
            fetch(idx_bufs[(g + 1) % 2], g + 1)
        if g > 0:
            wait_fetch(buf, g)
        gv, lv, _ = buf
        _agg_group(table, accum, gv, lv, rows0, rows1, semg0, semg1,
                   sems0, sems1, primed=(g == 0))

    plsc.subcore_barrier()
    @pl.when(s < _FULL_TILES)
    def _():
        pltpu.sync_copy(accum.at[pl.ds(s * _RPT, _RPT)],
                        out.at[c, pl.ds(s * _RPT, _RPT)])

    @pl.when(s == _FULL_TILES)
    def _():
        pltpu.sync_copy(accum.at[pl.ds(s * _RPT, _TAIL)],
                        out.at[c, pl.ds(s * _RPT, _TAIL)])


_agg = pl.kernel(
    _agg_body,
    mesh=_sc_mesh,
    out_type=jax.ShapeDtypeStruct((_NC, _NPAD, _D), jnp.float32),
    scratch_types=[
        pltpu.VMEM_SHARED((_NPAD, _D), jnp.float32),
        pltpu.VMEM((_GRP, _LANES), jnp.int32),
        pltpu.VMEM((_GRP, _LANES), jnp.int32),
        pltpu.VMEM((_GRP, _LANES), jnp.int32),
        pltpu.VMEM((_GRP, _LANES), jnp.int32),
        pltpu.VMEM((_LANES, _D), jnp.float32),
        pltpu.VMEM((_LANES, _D), jnp.float32),
        pltpu.SemaphoreType.DMA,
        pltpu.SemaphoreType.DMA,
        pltpu.SemaphoreType.DMA,
        pltpu.SemaphoreType.DMA,
        pltpu.SemaphoreType.DMA,
        pltpu.SemaphoreType.DMA,
        pltpu.SemaphoreType.DMA,
    ],
)


def _init_body(x_ref, wlin_ref, blin_ref, wetc_ref, betc_ref, h_ref, ht_ref):
    h = jnp.dot(x_ref[...], wlin_ref[...],
                preferred_element_type=jnp.float32) + blin_ref[...]
    h_ref[...] = h
    ht_ref[...] = jnp.dot(h, wetc_ref[...],
                          preferred_element_type=jnp.float32) + betc_ref[...]


def _gru_math(part_ref, h_ref, wih_ref, whh_ref, bih_ref, bhh_ref):
    a = part_ref[0] + part_ref[1]
    gi = jnp.dot(a, wih_ref[...], preferred_element_type=jnp.float32) + bih_ref[...]
    h = h_ref[...]
    gh = jnp.dot(h, whh_ref[...], preferred_element_type=jnp.float32) + bhh_ref[...]
    r = jax.nn.sigmoid(gi[:, :_D] + gh[:, :_D])
    z = jax.nn.sigmoid(gi[:, _D:2 * _D] + gh[:, _D:2 * _D])
    n = jnp.tanh(gi[:, 2 * _D:] + r * gh[:, 2 * _D:])
    return (1.0 - z) * n + z * h


def _gru_full_body(part_ref, h_ref, wih_ref, whh_ref, bih_ref,
                   bhh_ref, wetc_ref, betc_ref, hn_ref, ht_ref):
    hn = _gru_math(part_ref, h_ref, wih_ref, whh_ref, bih_ref, bhh_ref)
    hn_ref[...] = hn
    ht_ref[...] = jnp.dot(hn, wetc_ref[...],
                          preferred_element_type=jnp.float32) + betc_ref[...]


def _gru_last_body(part_ref, h_ref, wih_ref, whh_ref, bih_ref, bhh_ref,
                   wcls_ref, bcls_ref, out_ref, acc_ref):
    b = pl.program_id(0)
    hn = _gru_math(part_ref, h_ref, wih_ref, whh_ref, bih_ref, bhh_ref)

    @pl.when(b == 0)
    def _():
        acc_ref[...] = jnp.zeros_like(acc_ref)

    acc_ref[...] += jnp.sum(hn.reshape(_B // 8, 8, _D), axis=0)

    @pl.when(b == _NB - 1)
    def _():
        feats = jnp.sum(acc_ref[...], axis=0, keepdims=True)
        out_ref[...] = jnp.dot(feats, wcls_ref[...],
                               preferred_element_type=jnp.float32) + bcls_ref[...]


def _blk(shape, index_map):
    return pl.BlockSpec(shape, index_map)


_init_call = pl.pallas_call(
    _init_body,
    grid=(_NB,),
    in_specs=[
        _blk((_B, _D), lambda b: (b, 0)),
        _blk((_D, _D), lambda b: (0, 0)),
        _blk((1, _D), lambda b: (0, 0)),
        _blk((_D, _T * _D), lambda b: (0, 0)),
        _blk((1, _T * _D), lambda b: (0, 0)),
    ],
    out_specs=[
        _blk((_B, _D), lambda b: (b, 0)),
        _blk((_B, _T * _D), lambda b: (b, 0)),
    ],
    out_shape=[
        jax.ShapeDtypeStruct((_N, _D), jnp.float32),
        jax.ShapeDtypeStruct((_N, _T * _D), jnp.float32),
    ],
)

_gru_in_specs = [
    _blk((_NC, _B, _D), lambda b: (0, b, 0)),
    _blk((_B, _D), lambda b: (b, 0)),
    _blk((_D, 3 * _D), lambda b: (0, 0)),
    _blk((_D, 3 * _D), lambda b: (0, 0)),
    _blk((1, 3 * _D), lambda b: (0, 0)),
    _blk((1, 3 * _D), lambda b: (0, 0)),
]

_gru_full_call = pl.pallas_call(
    _gru_full_body,
    grid=(_NB,),
    in_specs=_gru_in_specs + [
        _blk((_D, _T * _D), lambda b: (0, 0)),
        _blk((1, _T * _D), lambda b: (0, 0)),
    ],
    out_specs=[
        _blk((_B, _D), lambda b: (b, 0)),
        _blk((_B, _T * _D), lambda b: (b, 0)),
    ],
    out_shape=[
        jax.ShapeDtypeStruct((_N, _D), jnp.float32),
        jax.ShapeDtypeStruct((_N, _T * _D), jnp.float32),
    ],
)

_gru_last_call = pl.pallas_call(
    _gru_last_body,
    grid=(_NB,),
    in_specs=_gru_in_specs + [
        _blk((_D, _D), lambda b: (0, 0)),
        _blk((1, _D), lambda b: (0, 0)),
    ],
    out_specs=_blk((1, _D), lambda b: (0, 0)),
    out_shape=jax.ShapeDtypeStruct((1, _D), jnp.float32),
    scratch_shapes=[pltpu.VMEM((8, _D), jnp.float32)],
)


def kernel(x, edge_index, edge_type, W_lin, b_lin, W_et, b_et, W_ih, W_hh,
           b_ih, b_hh, W_cls, b_cls):
    src = edge_index[0]
    dst = edge_index[1]

    ppw = _CHUNKS * _LANES - _E // _NW
    jj = jnp.arange(ppw, dtype=jnp.int32)[None, :]
    ww = jnp.arange(_NW, dtype=jnp.int32)[:, None]
    gpad = (ww * 10007 + jj * 263) % (_T * _N)
    lpad = _N + (ww % _NS) * 71 + (jj % 71)
    gidx = jnp.concatenate([(src * _T + edge_type).reshape(_NW, -1), gpad],
                           axis=1).reshape(_NW, _CHUNKS, _LANES)
    ldst = jnp.concatenate([dst.reshape(_NW, -1), lpad],
                           axis=1).reshape(_NW, _CHUNKS, _LANES)
    zeros = jnp.zeros((_NPAD, _D), jnp.float32)

    blin = b_lin.reshape(1, _D)
    wetc = jnp.transpose(W_et, (1, 0, 2)).reshape(_D, _T * _D)
    betc = b_et.reshape(1, _T * _D)
    wih_t = W_ih.T
    whh_t = W_hh.T
    bih = b_ih.reshape(1, 3 * _D)
    bhh = b_hh.reshape(1, 3 * _D)
    wcls_pad = jnp.zeros((_D, _D), jnp.float32).at[:, :2].set(W_cls)
    bcls_pad = jnp.zeros((1, _D), jnp.float32).at[0, :2].set(b_cls)

    h, ht = _init_call(x, W_lin, blin, wetc, betc)

    out = None
    for step in range(_STEPS):
        part = _agg(ht.reshape(_T * _N, _D), gidx, ldst, zeros)
        if step < _STEPS - 1:
            h, ht = _gru_full_call(part, h, wih_t, whh_t, bih, bhh, wetc, betc)
        else:
            out = _gru_last_call(part, h, wih_t, whh_t, bih, bhh,
                                 wcls_pad, bcls_pad)

    return out[:, :2]


# speedup vs baseline: 1.1155x; 1.1155x over previous
import jax
import jax.numpy as jnp
from jax import lax
from jax.experimental import pallas as pl
from jax.experimental.pallas import tpu as pltpu
from jax.experimental.pallas import tpu_sc as plsc

_N = 10000
_E = 320000
_D = 128
_T = 4
_STEPS = 8

_NC = 2
_NS = 16
_NW = _NC * _NS
_LANES = 128
_CHUNKS = 80
_EPAD = _NW * _CHUNKS * _LANES
_NPAD = 11136
_RPT = _NPAD // _NS
_GRP = 16
_NGRP = _CHUNKS // _GRP
_FULL_TILES = _N // _RPT
_TAIL = _N - _FULL_TILES * _RPT

_B = 1000
_NB = _N // _B


_sc_mesh = plsc.VectorSubcoreMesh(core_axis_name="c", subcore_axis_name="s")


def _agg_group(table, accum, gv, lv, rows0, rows1, semg0, semg1, sems0, sems1,
               primed):
    if not primed:
        pltpu.async_copy(table.at[gv.at[0]], rows0, semg0)
        pltpu.async_copy(table.at[gv.at[1]], rows1, semg1)

    def body(k, carry):
        j0 = 2 * k
        j1 = j0 + 1
        jn0 = jnp.minimum(j0 + 2, _GRP - 2)
        jn1 = jnp.minimum(j1 + 2, _GRP - 1)
        pltpu.make_async_copy(table.at[gv.at[j0]], rows0, semg0).wait()
        pltpu.async_copy(rows0, accum.at[lv.at[j0]], sems0, add=True)
        pltpu.make_async_copy(table.at[gv.at[j1]], rows1, semg1).wait()
        pltpu.async_copy(rows1, accum.at[lv.at[j1]], sems1, add=True)
        pltpu.make_async_copy(rows0, accum.at[lv.at[j0]], sems0).wait()
        pltpu.async_copy(table.at[gv.at[jn0]], rows0, semg0)
        pltpu.make_async_copy(rows1, accum.at[lv.at[j1]], sems1).wait()
        pltpu.async_copy(table.at[gv.at[jn1]], rows1, semg1)
        return carry

    lax.fori_loop(0, _GRP // 2, body, 0)
    pltpu.make_async_copy(table.at[gv.at[_GRP - 2]], rows0, semg0).wait()
    pltpu.make_async_copy(table.at[gv.at[_GRP - 1]], rows1, semg1).wait()


def _agg_body(table, gidx, ldst, zeros, out, accum, gvA, lvA, gvB, lvB,
              rows0, rows1, semg0, semg1, sems0, sems1, semA, semB, semz):
    c = lax.axis_index("c")
    s = lax.axis_index("s")
    w = c * _NS + s

    pltpu.async_copy(zeros.at[pl.ds(s * _RPT, _RPT)],
                     accum.at[pl.ds(s * _RPT, _RPT)], semz)

    idx_bufs = [(gvA, lvA, semA), (gvB, lvB, semB)]

    def fetch(buf, g):
        gv, lv, sem = buf
        pltpu.async_copy(gidx.at[w, pl.ds(g * _GRP, _GRP)], gv, sem)
        pltpu.async_copy(ldst.at[w, pl.ds(g * _GRP, _GRP)], lv, sem)

    def wait_fetch(buf, g):
        gv, lv, sem = buf
        pltpu.make_async_copy(gidx.at[w, pl.ds(g * _GRP, _GRP)], gv, sem).wait()
        pltpu.make_async_copy(ldst.at[w, pl.ds(g * _GRP, _GRP)], lv, sem).wait()

    fetch(idx_bufs[0], 0)
    wait_fetch(idx_bufs[0], 0)
    pltpu.async_copy(table.at[gvA.at[0]], rows0, semg0)
    pltpu.async_copy(table.at[gvA.at[1]], rows1, semg1)
    pltpu.make_async_copy(zeros.at[pl.ds(s * _RPT, _RPT)],
                          accum.at[pl.ds(s * _RPT, _RPT)], semz).wait()
    plsc.subcore_barrier()

    for g in range(_NGRP):
        buf = idx_bufs[g % 2]
        if g + 1 < _NGRP:
            fetch(idx_bufs[(g + 1) % 2], g + 1)
        if g > 0:
            wait_fetch(buf, g)
        gv, lv, _ = buf
        _agg_group(table, accum, gv, lv, rows0, rows1, semg0, semg1,
                   sems0, sems1, primed=(g == 0))

    plsc.subcore_barrier()
    @pl.when(s < _FULL_TILES)
    def _():
        pltpu.sync_copy(accum.at[pl.ds(s * _RPT, _RPT)],
                        out.at[c, pl.ds(s * _RPT, _RPT)])

    @pl.when(s == _FULL_TILES)
    def _():
        pltpu.sync_copy(accum.at[pl.ds(s * _RPT, _TAIL)],
                        out.at[c, pl.ds(s * _RPT, _TAIL)])


_agg = pl.kernel(
    _agg_body,
    mesh=_sc_mesh,
    out_type=jax.ShapeDtypeStruct((_NC, _NPAD, _D), jnp.float32),
    scratch_types=[
        pltpu.VMEM_SHARED((_NPAD, _D), jnp.float32),
        pltpu.VMEM((_GRP, _LANES), jnp.int32),
        pltpu.VMEM((_GRP, _LANES), jnp.int32),
        pltpu.VMEM((_GRP, _LANES), jnp.int32),
        pltpu.VMEM((_GRP, _LANES), jnp.int32),
        pltpu.VMEM((_LANES, _D), jnp.float32),
        pltpu.VMEM((_LANES, _D), jnp.float32),
        pltpu.SemaphoreType.DMA,
        pltpu.SemaphoreType.DMA,
        pltpu.SemaphoreType.DMA,
        pltpu.SemaphoreType.DMA,
        pltpu.SemaphoreType.DMA,
        pltpu.SemaphoreType.DMA,
        pltpu.SemaphoreType.DMA,
    ],
)


def _init_body(x_ref, wlin_ref, blin_ref, wet_ref, bet_ref, h_ref, ht_ref):
    h = jnp.dot(x_ref[...], wlin_ref[...],
                preferred_element_type=jnp.float32) + blin_ref[...]
    h_ref[...] = h
    for t in range(_T):
        ht_ref[t] = jnp.dot(h, wet_ref[t],
                            preferred_element_type=jnp.float32) + bet_ref[t]


def _gru_math(part_ref, h_ref, wih_ref, whh_ref, bih_ref, bhh_ref):
    a = part_ref[0] + part_ref[1]
    gi = jnp.dot(a, wih_ref[...], preferred_element_type=jnp.float32) + bih_ref[...]
    h = h_ref[...]
    gh = jnp.dot(h, whh_ref[...], preferred_element_type=jnp.float32) + bhh_ref[...]
    r = jax.nn.sigmoid(gi[:, :_D] + gh[:, :_D])
    z = jax.nn.sigmoid(gi[:, _D:2 * _D] + gh[:, _D:2 * _D])
    n = jnp.tanh(gi[:, 2 * _D:] + r * gh[:, 2 * _D:])
    return (1.0 - z) * n + z * h


def _gru_full_body(part_ref, h_ref, wih_ref, whh_ref, bih_ref,
                   bhh_ref, wet_ref, bet_ref, hn_ref, ht_ref):
    hn = _gru_math(part_ref, h_ref, wih_ref, whh_ref, bih_ref, bhh_ref)
    hn_ref[...] = hn
    for t in range(_T):
        ht_ref[t] = jnp.dot(hn, wet_ref[t],
                            preferred_element_type=jnp.float32) + bet_ref[t]


def _gru_last_body(part_ref, h_ref, wih_ref, whh_ref, bih_ref, bhh_ref,
                   wcls_ref, bcls_ref, out_ref, acc_ref):
    b = pl.program_id(0)
    hn = _gru_math(part_ref, h_ref, wih_ref, whh_ref, bih_ref, bhh_ref)

    @pl.when(b == 0)
    def _():
        acc_ref[...] = jnp.zeros_like(acc_ref)

    acc_ref[...] += jnp.sum(hn.reshape(_B // 8, 8, _D), axis=0)

    @pl.when(b == _NB - 1)
    def _():
        feats = jnp.sum(acc_ref[...], axis=0, keepdims=True)
        out_ref[...] = jnp.dot(feats, wcls_ref[...],
                               preferred_element_type=jnp.float32) + bcls_ref[...]


def _blk(shape, index_map):
    return pl.BlockSpec(shape, index_map)


_init_call = pl.pallas_call(
    _init_body,
    grid=(_NB,),
    in_specs=[
        _blk((_B, _D), lambda b: (b, 0)),
        _blk((_D, _D), lambda b: (0, 0)),
        _blk((1, _D), lambda b: (0, 0)),
        _blk((_T, _D, _D), lambda b: (0, 0, 0)),
        _blk((_T, 1, _D), lambda b: (0, 0, 0)),
    ],
    out_specs=[
        _blk((_B, _D), lambda b: (b, 0)),
        _blk((_T, _B, _D), lambda b: (0, b, 0)),
    ],
    out_shape=[
        jax.ShapeDtypeStruct((_N, _D), jnp.float32),
        jax.ShapeDtypeStruct((_T, _N, _D), jnp.float32),
    ],
)

_gru_in_specs = [
    _blk((_NC, _B, _D), lambda b: (0, b, 0)),
    _blk((_B, _D), lambda b: (b, 0)),
    _blk((_D, 3 * _D), lambda b: (0, 0)),
    _blk((_D, 3 * _D), lambda b: (0, 0)),
    _blk((1, 3 * _D), lambda b: (0, 0)),
    _blk((1, 3 * _D), lambda b: (0, 0)),
]

_gru_full_call = pl.pallas_call(
    _gru_full_body,
    grid=(_NB,),
    in_specs=_gru_in_specs + [
        _blk((_T, _D, _D), lambda b: (0, 0, 0)),
        _blk((_T, 1, _D), lambda b: (0, 0, 0)),
    ],
    out_specs=[
        _blk((_B, _D), lambda b: (b, 0)),
        _blk((_T, _B, _D), lambda b: (0, b, 0)),
    ],
    out_shape=[
        jax.ShapeDtypeStruct((_N, _D), jnp.float32),
        jax.ShapeDtypeStruct((_T, _N, _D), jnp.float32),
    ],
)

_gru_last_call = pl.pallas_call(
    _gru_last_body,
    grid=(_NB,),
    in_specs=_gru_in_specs + [
        _blk((_D, _D), lambda b: (0, 0)),
        _blk((1, _D), lambda b: (0, 0)),
    ],
    out_specs=_blk((1, _D), lambda b: (0, 0)),
    out_shape=jax.ShapeDtypeStruct((1, _D), jnp.float32),
    scratch_shapes=[pltpu.VMEM((8, _D), jnp.float32)],
)


def kernel(x, edge_index, edge_type, W_lin, b_lin, W_et, b_et, W_ih, W_hh,
           b_ih, b_hh, W_cls, b_cls):
    src = edge_index[0]
    dst = edge_index[1]

    ppw = _CHUNKS * _LANES - _E // _NW
    jj = jnp.arange(ppw, dtype=jnp.int32)[None, :]
    ww = jnp.arange(_NW, dtype=jnp.int32)[:, None]
    gpad = (ww * 10007 + jj * 263) % (_T * _N)
    lpad = _N + (ww % _NS) * 71 + (jj % 71)
    gidx = jnp.concatenate([(edge_type * _N + src).reshape(_NW, -1), gpad],
                           axis=1).reshape(_NW, _CHUNKS, _LANES)
    ldst = jnp.concatenate([dst.reshape(_NW, -1), lpad],
                           axis=1).reshape(_NW, _CHUNKS, _LANES)
    zeros = jnp.zeros((_NPAD, _D), jnp.float32)

    blin = b_lin.reshape(1, _D)
    bet = b_et.reshape(_T, 1, _D)
    wih_t = W_ih.T
    whh_t = W_hh.T
    bih = b_ih.reshape(1, 3 * _D)
    bhh = b_hh.reshape(1, 3 * _D)
    wcls_pad = jnp.zeros((_D, _D), jnp.float32).at[:, :2].set(W_cls)
    bcls_pad = jnp.zeros((1, _D), jnp.float32).at[0, :2].set(b_cls)

    h, ht = _init_call(x, W_lin, blin, W_et, bet)

    out = None
    for step in range(_STEPS):
        part = _agg(ht.reshape(_T * _N, _D), gidx, ldst, zeros)
        if step < _STEPS - 1:
            h, ht = _gru_full_call(part, h, wih_t, whh_t, bih, bhh, W_et, bet)
        else:
            out = _gru_last_call(part, h, wih_t, whh_t, bih, bhh,
                                 wcls_pad, bcls_pad)

    return out[:, :2]
